# baseline (speedup 1.0000x reference)
import jax
import jax.numpy as jnp
from jax.experimental import pallas as pl
from jax.experimental.pallas import tpu as pltpu

_LN_EPS = 1e-5
_LEAKY_SLOPE = 0.01
_NEG_BIG = -1e30


def _gat_tile_kernel(x_ref, gamma_ref, beta_ref, wqkv_ref, bqkv_ref, o_ref):
    bt, J, D = x_ref.shape
    M = bt * J

    x = x_ref[...].reshape(M, D)

    mu = jnp.mean(x, axis=-1, keepdims=True)
    xc = x - mu
    var = jnp.mean(xc * xc, axis=-1, keepdims=True)
    xn = xc * jax.lax.rsqrt(var + _LN_EPS)
    xn = xn * gamma_ref[...] + beta_ref[...]

    qkv = jnp.dot(xn, wqkv_ref[...], preferred_element_type=jnp.float32)
    qkv = qkv + bqkv_ref[...]
    q = qkv[:, :D]
    k = qkv[:, D:2 * D]
    v = qkv[:, 2 * D:]

    s = jax.lax.dot_general(q, k, (((1,), (1,)), ((), ())),
                            preferred_element_type=jnp.float32)

    ri = jax.lax.broadcasted_iota(jnp.int32, (M, M), 0)
    ci = jax.lax.broadcasted_iota(jnp.int32, (M, M), 1)
    same = (ri // J) == (ci // J)
    s = jnp.where(same, s, _NEG_BIG)

    s = s - jnp.max(s, axis=-1, keepdims=True)
    p = jnp.exp(s)
    w = p / jnp.sum(p, axis=-1, keepdims=True)

    att = jnp.dot(w, v, preferred_element_type=jnp.float32)

    act = jnp.where(att >= 0, att, _LEAKY_SLOPE * att)
    o_ref[...] = (act + x).reshape(bt, J, D).astype(o_ref.dtype)


def kernel(x, gamma, beta, wqkv, bqkv):
    B, J, D = x.shape
    BT = 8
    grid_b = B // BT

    fixed = lambda b: (0, 0)

    return pl.pallas_call(
        _gat_tile_kernel,
        out_shape=jax.ShapeDtypeStruct((B, J, D), x.dtype),
        grid=(grid_b,),
        in_specs=[
            pl.BlockSpec((BT, J, D), lambda b: (b, 0, 0)),
            pl.BlockSpec((1, D), fixed),
            pl.BlockSpec((1, D), fixed),
            pl.BlockSpec((D, 3 * D), fixed),
            pl.BlockSpec((1, 3 * D), fixed),
        ],
        out_specs=pl.BlockSpec((BT, J, D), lambda b: (b, 0, 0)),
        compiler_params=pltpu.CompilerParams(
            dimension_semantics=("parallel",)),
    )(x, gamma, beta, wqkv, bqkv)


# speedup vs baseline: 2.8744x; 2.8744x over previous
import jax
import jax.numpy as jnp
from jax.experimental import pallas as pl
from jax.experimental.pallas import tpu as pltpu

_LN_EPS = 1e-5
_LEAKY_SLOPE = 0.01
_NEG_BIG = -1e30


_CHUNK = 256


def _gat_tile_kernel(x_ref, gamma_ref, beta_ref, wqkv_ref, bqkv_ref, o_ref):
    bt, J, D = x_ref.shape
    M = bt * J
    G = M // _CHUNK

    x = x_ref[...].reshape(M, D)

    mu = jnp.mean(x, axis=-1, keepdims=True)
    xc = x - mu
    var = jnp.mean(xc * xc, axis=-1, keepdims=True)
    xn = xc * jax.lax.rsqrt(var + _LN_EPS)
    xn = xn * gamma_ref[...] + beta_ref[...]

    qkv = jnp.dot(xn, wqkv_ref[...], preferred_element_type=jnp.float32)
    qkv = qkv + bqkv_ref[...]
    q = qkv[:, :D].reshape(G, _CHUNK, D)
    k = qkv[:, D:2 * D].reshape(G, _CHUNK, D)
    v = qkv[:, 2 * D:].reshape(G, _CHUNK, D)

    s = jax.lax.dot_general(q, k, (((2,), (2,)), ((0,), (0,))),
                            preferred_element_type=jnp.float32)

    ri = jax.lax.broadcasted_iota(jnp.int32, (_CHUNK, _CHUNK), 0)
    ci = jax.lax.broadcasted_iota(jnp.int32, (_CHUNK, _CHUNK), 1)
    same = (ri // J) == (ci // J)
    s = jnp.where(same[None], s, _NEG_BIG)

    s = s - jnp.max(s, axis=-1, keepdims=True)
    p = jnp.exp(s)
    w = p / jnp.sum(p, axis=-1, keepdims=True)

    att = jax.lax.dot_general(w, v, (((2,), (1,)), ((0,), (0,))),
                              preferred_element_type=jnp.float32)

    act = jnp.where(att >= 0, att, _LEAKY_SLOPE * att)
    o_ref[...] = (act.reshape(M, D) + x).reshape(bt, J, D).astype(o_ref.dtype)


def kernel(x, gamma, beta, wqkv, bqkv):
    B, J, D = x.shape
    BT = 64
    grid_b = B // BT

    fixed = lambda b: (0, 0)

    return pl.pallas_call(
        _gat_tile_kernel,
        out_shape=jax.ShapeDtypeStruct((B, J, D), x.dtype),
        grid=(grid_b,),
        in_specs=[
            pl.BlockSpec((BT, J, D), lambda b: (b, 0, 0)),
            pl.BlockSpec((1, D), fixed),
            pl.BlockSpec((1, D), fixed),
            pl.BlockSpec((D, 3 * D), fixed),
            pl.BlockSpec((1, 3 * D), fixed),
        ],
        out_specs=pl.BlockSpec((BT, J, D), lambda b: (b, 0, 0)),
        compiler_params=pltpu.CompilerParams(
            dimension_semantics=("parallel",)),
    )(x, gamma, beta, wqkv, bqkv)


# speedup vs baseline: 3.1691x; 1.1025x over previous
import jax
import jax.numpy as jnp
from jax.experimental import pallas as pl
from jax.experimental.pallas import tpu as pltpu

_LN_EPS = 1e-5
_LEAKY_SLOPE = 0.01


def _gat_tile_kernel(x_ref, gamma_ref, beta_ref, wqkv_ref, bqkv_ref, o_ref):
    bt, J, D = x_ref.shape
    M = bt * J

    x = x_ref[...].reshape(M, D)

    mu = jnp.mean(x, axis=-1, keepdims=True)
    xc = x - mu
    var = jnp.mean(xc * xc, axis=-1, keepdims=True)
    xn = xc * jax.lax.rsqrt(var + _LN_EPS)
    xn = xn * gamma_ref[...] + beta_ref[...]

    qkv = jnp.dot(xn, wqkv_ref[...], preferred_element_type=jnp.float32)
    qkv = qkv + bqkv_ref[...]
    qkv = qkv.reshape(bt, J, 3 * D)
    q = qkv[..., :D]
    k = qkv[..., D:2 * D]
    v = qkv[..., 2 * D:]

    s = jnp.einsum("bqd,bkd->bqk", q, k,
                   preferred_element_type=jnp.float32)
    p = jnp.exp(s)
    r = 1.0 / jnp.sum(p, axis=-1, keepdims=True)
    att = jnp.einsum("bqk,bkd->bqd", p, v,
                     preferred_element_type=jnp.float32)
    att = att * r

    act = jnp.where(att >= 0, att, _LEAKY_SLOPE * att)
    o_ref[...] = (act.reshape(M, D) + x).reshape(bt, J, D).astype(o_ref.dtype)


def kernel(x, gamma, beta, wqkv, bqkv):
    B, J, D = x.shape
    BT = 64
    grid_b = B // BT

    fixed = lambda b: (0, 0)

    return pl.pallas_call(
        _gat_tile_kernel,
        out_shape=jax.ShapeDtypeStruct((B, J, D), x.dtype),
        grid=(grid_b,),
        in_specs=[
            pl.BlockSpec((BT, J, D), lambda b: (b, 0, 0)),
            pl.BlockSpec((1, D), fixed),
            pl.BlockSpec((1, D), fixed),
            pl.BlockSpec((D, 3 * D), fixed),
            pl.BlockSpec((1, 3 * D), fixed),
        ],
        out_specs=pl.BlockSpec((BT, J, D), lambda b: (b, 0, 0)),
        compiler_params=pltpu.CompilerParams(
            dimension_semantics=("parallel",)),
    )(x, gamma, beta, wqkv, bqkv)
